# baseline (speedup 1.0000x reference)
import jax
import jax.numpy as jnp
from jax.experimental import pallas as pl

_N = 8192
_B = 8
_K = 20
_RT = 256


def _knn_idx(x, batch, k):
    sq = jnp.sum(x * x, axis=1)
    d2 = sq[:, None] + sq[None, :] - 2.0 * (x @ x.T)
    mask = batch[:, None] != batch[None, :]
    d2 = jnp.where(mask, jnp.inf, d2)
    _, idx = jax.lax.top_k(-d2, k)
    return idx


def _aggr_pool_body(x_ref, batch_ref, w_ref, b_ref, out_ref):
    t = pl.program_id(0)
    h = x_ref[...] @ w_ref[...] + b_ref[...][None, :]
    bt = batch_ref[0]
    rows = []
    for b in range(_B):
        m = bt == b
        rows.append(jnp.max(jnp.where(m, h, -jnp.inf), axis=0, keepdims=True))
    acc = jnp.concatenate(rows, axis=0)

    @pl.when(t == 0)
    def _():
        out_ref[...] = acc

    @pl.when(t != 0)
    def _():
        out_ref[...] = jnp.maximum(out_ref[...], acc)


def kernel(pos, batch, b0l0_W, b0l0_b, b0l1_W, b0l1_b, b1l0_W, b1l0_b,
           aggr_W, aggr_b, h0_W, h0_b, h1_W, h1_b, h2_W, h2_b):
    batch = batch.astype(jnp.int32)

    idx1 = _knn_idx(pos, batch, _K)
    a1 = pos @ b0l0_W[:3] + b0l0_b
    bm1 = pos @ b0l0_W[3:]
    c1 = a1 - bm1
    e = c1[:, None, :] + bm1[idx1]
    h = jax.nn.relu(e) @ b0l1_W + b0l1_b
    x1 = jnp.max(h, axis=1)

    idx2 = _knn_idx(x1, batch, _K)
    c2 = x1 @ (b1l0_W[:64] - b1l0_W[64:]) + b1l0_b
    d2 = x1 @ b1l0_W[64:]
    x2 = c2 + jnp.max(d2[idx2], axis=1)

    xcat = jnp.concatenate([x1, x2], axis=1)
    bt2 = batch.reshape(_N // _RT, _RT, 1)
    pooled = pl.pallas_call(
        _aggr_pool_body,
        grid=(_N // _RT,),
        in_specs=[
            pl.BlockSpec((_RT, 192), lambda t: (t, 0)),
            pl.BlockSpec((1, _RT, 1), lambda t: (t, 0, 0)),
            pl.BlockSpec((192, 1024), lambda t: (0, 0)),
            pl.BlockSpec((1024,), lambda t: (0,)),
        ],
        out_specs=pl.BlockSpec((_B, 1024), lambda t: (0, 0)),
        out_shape=jax.ShapeDtypeStruct((_B, 1024), jnp.float32),
    )(xcat, bt2, aggr_W, aggr_b)

    h = jax.nn.relu(pooled @ h0_W + h0_b)
    h = jax.nn.relu(h @ h1_W + h1_b)
    return h @ h2_W + h2_b


# speedup vs baseline: 4.3933x; 4.3933x over previous
import functools

import jax
import jax.numpy as jnp
from jax.experimental import pallas as pl
from jax.experimental.pallas import tpu as pltpu

_N = 8192
_B = 8
_K = 20
_RT = 256
_CT = 256


def _knn_body(se_ref, cb_ref, xr_ref, btr_ref, xc_ref, btc_ref, idx_ref,
              strip_ref, *, d):
    t = pl.program_id(0)
    c0 = cb_ref[t, 0]
    c1 = cb_ref[t, 1]

    rr = xr_ref[...]
    sq_r = jnp.sum(rr * rr, axis=1, keepdims=True)
    aug = jnp.concatenate([-2.0 * rr, jnp.ones((_RT, 1), jnp.float32)], 1)
    btr = btr_ref[...]

    def dist_tile(c, _):
        cc = xc_ref[pl.ds(c * _CT, _CT), :]
        sqc = jnp.sum(cc * cc, axis=1, keepdims=True)
        bmat = jnp.concatenate([cc, sqc], 1)
        d2 = jax.lax.dot_general(aug, bmat, (((1,), (1,)), ((), ())),
                                 preferred_element_type=jnp.float32)
        d2 = d2 + sq_r
        btc = btc_ref[:, pl.ds(c * _CT, _CT)]
        d2 = jnp.where(btr != btc, jnp.inf, d2)
        strip_ref[:, pl.ds(c * _CT, _CT)] = d2
        return 0

    jax.lax.fori_loop(c0, c1, dist_tile, 0)

    s_row = jnp.zeros((_RT, 1), jnp.int32)
    e_row = jnp.zeros((_RT, 1), jnp.int32)
    for b in range(_B):
        s_row = jnp.where(btr == b, se_ref[0, b], s_row)
        e_row = jnp.where(btr == b, se_ref[1, b], e_row)
    nvalid = e_row - s_row

    iota = jax.lax.broadcasted_iota(jnp.int32, (_RT, _CT), 1)
    inf = jnp.float32(jnp.inf)
    big = jnp.int32(2 * _N)

    prev = jnp.full((_RT, 1), -1, jnp.int32)
    for k in range(_K):
        def minpass(c, m):
            col = jax.lax.mul(c, _CT)
            tile = strip_ref[:, pl.ds(col, _CT)]
            tile = jnp.where(iota + col == prev, inf, tile)
            strip_ref[:, pl.ds(col, _CT)] = tile
            return jnp.minimum(m, jnp.min(tile, axis=1, keepdims=True))

        m = jax.lax.fori_loop(c0, c1, minpass, jnp.full((_RT, 1), inf))

        def argpass(c, idx):
            col = jax.lax.mul(c, _CT)
            tile = strip_ref[:, pl.ds(col, _CT)]
            cand = jnp.where(tile == m, iota + col, big)
            return jnp.minimum(idx, jnp.min(cand, axis=1, keepdims=True))

        idx = jax.lax.fori_loop(c0, c1, argpass, jnp.full((_RT, 1), big))

        p = k - nvalid
        idxfix = jnp.where(p < s_row, p, e_row + (p - s_row))
        exhausted = m == inf
        idx = jnp.where(exhausted, idxfix, idx)
        idx_ref[:, k:k + 1] = idx
        prev = idx


def _knn(x, btr2, btc2, se, cb, d):
    n = x.shape[0]
    grid_spec = pltpu.PrefetchScalarGridSpec(
        num_scalar_prefetch=2,
        grid=(n // _RT,),
        in_specs=[
            pl.BlockSpec((_RT, d), lambda t, se, cb: (t, 0)),
            pl.BlockSpec((_RT, 1), lambda t, se, cb: (t, 0)),
            pl.BlockSpec((n, d), lambda t, se, cb: (0, 0)),
            pl.BlockSpec((1, n), lambda t, se, cb: (0, 0)),
        ],
        out_specs=pl.BlockSpec((_RT, _K), lambda t, se, cb: (t, 0)),
        scratch_shapes=[pltpu.VMEM((_RT, n), jnp.float32)],
    )
    return pl.pallas_call(
        functools.partial(_knn_body, d=d),
        grid_spec=grid_spec,
        out_shape=jax.ShapeDtypeStruct((n, _K), jnp.int32),
    )(se, cb, x, btr2, x, btc2)


def _aggr_pool_body(x_ref, batch_ref, w_ref, b_ref, out_ref):
    t = pl.program_id(0)
    h = x_ref[...] @ w_ref[...] + b_ref[...][None, :]
    bt = batch_ref[0]
    rows = []
    for b in range(_B):
        m = bt == b
        rows.append(jnp.max(jnp.where(m, h, -jnp.inf), axis=0, keepdims=True))
    acc = jnp.concatenate(rows, axis=0)

    @pl.when(t == 0)
    def _():
        out_ref[...] = acc

    @pl.when(t != 0)
    def _():
        out_ref[...] = jnp.maximum(out_ref[...], acc)


def kernel(pos, batch, b0l0_W, b0l0_b, b0l1_W, b0l1_b, b1l0_W, b1l0_b,
           aggr_W, aggr_b, h0_W, h0_b, h1_W, h1_b, h2_W, h2_b):
    batch = batch.astype(jnp.int32)

    arangeb = jnp.arange(_B, dtype=jnp.int32)
    starts = jnp.searchsorted(batch, arangeb, side="left").astype(jnp.int32)
    ends = jnp.searchsorted(batch, arangeb, side="right").astype(jnp.int32)
    se = jnp.stack([starts, ends])
    btile = batch.reshape(_N // _RT, _RT)
    c0 = starts[btile[:, 0]] // _CT
    c1 = (ends[btile[:, -1]] + _CT - 1) // _CT
    cb = jnp.stack([c0, c1], axis=1).astype(jnp.int32)
    btr2 = batch.reshape(_N, 1)
    btc2 = batch.reshape(1, _N)

    pos8 = jnp.pad(pos, ((0, 0), (0, 5)))
    idx1 = _knn(pos8, btr2, btc2, se, cb, 8)
    a1 = pos @ b0l0_W[:3] + b0l0_b
    bm1 = pos @ b0l0_W[3:]
    c1_ = a1 - bm1
    e = c1_[:, None, :] + bm1[idx1]
    h = jax.nn.relu(e) @ b0l1_W + b0l1_b
    x1 = jnp.max(h, axis=1)

    idx2 = _knn(x1, btr2, btc2, se, cb, 64)
    c2 = x1 @ (b1l0_W[:64] - b1l0_W[64:]) + b1l0_b
    d2v = x1 @ b1l0_W[64:]
    x2 = c2 + jnp.max(d2v[idx2], axis=1)

    xcat = jnp.concatenate([x1, x2], axis=1)
    bt3 = batch.reshape(_N // _RT, _RT, 1)
    pooled = pl.pallas_call(
        _aggr_pool_body,
        grid=(_N // _RT,),
        in_specs=[
            pl.BlockSpec((_RT, 192), lambda t: (t, 0)),
            pl.BlockSpec((1, _RT, 1), lambda t: (t, 0, 0)),
            pl.BlockSpec((192, 1024), lambda t: (0, 0)),
            pl.BlockSpec((1024,), lambda t: (0,)),
        ],
        out_specs=pl.BlockSpec((_B, 1024), lambda t: (0, 0)),
        out_shape=jax.ShapeDtypeStruct((_B, 1024), jnp.float32),
    )(xcat, bt3, aggr_W, aggr_b)

    h = jax.nn.relu(pooled @ h0_W + h0_b)
    h = jax.nn.relu(h @ h1_W + h1_b)
    return h @ h2_W + h2_b


# speedup vs baseline: 9.8628x; 2.2450x over previous
import functools

import jax
import jax.numpy as jnp
from jax.experimental import pallas as pl
from jax.experimental.pallas import tpu as pltpu

_N = 8192
_B = 8
_K = 20
_RT = 256
_CT = 256


def _knn_body(se_ref, cb_ref, xr_ref, btr_ref, xc_ref, btc_ref, idx_ref,
              strip_ref, cmin_ref, *, d):
    t = pl.program_id(0)
    c0 = cb_ref[t, 0]
    c1 = cb_ref[t, 1]
    nt = strip_ref.shape[0] // _CT

    rr = xr_ref[...]
    sq_r = jnp.sum(rr * rr, axis=1, keepdims=True)
    brmat = jnp.concatenate(
        [rr, jnp.ones((_RT, 1), jnp.float32), sq_r], 1)
    btr = btr_ref[...]

    inf = jnp.float32(jnp.inf)
    big = jnp.int32(2 * _N)
    cmin_ref[...] = jnp.full(cmin_ref.shape, inf)

    def dist_tile(c, _):
        cc = xc_ref[pl.ds(c * _CT, _CT), :]
        sqc = jnp.sum(cc * cc, axis=1, keepdims=True)
        acmat = jnp.concatenate(
            [-2.0 * cc, sqc, jnp.ones((_CT, 1), jnp.float32)], 1)
        d2 = jax.lax.dot_general(acmat, brmat, (((1,), (1,)), ((), ())),
                                 preferred_element_type=jnp.float32)
        btc = btc_ref[pl.ds(c * _CT, _CT), :]
        d2 = jnp.where(btc != btr, inf, d2)
        strip_ref[pl.ds(c * _CT, _CT), :] = d2
        cmin_ref[pl.ds(c, 1), :] = jnp.min(d2, axis=0, keepdims=True)
        return 0

    jax.lax.fori_loop(c0, c1, dist_tile, 0)

    s_row = jnp.zeros((1, _RT), jnp.int32)
    e_row = jnp.zeros((1, _RT), jnp.int32)
    for b in range(_B):
        s_row = jnp.where(btr == b, se_ref[0, b], s_row)
        e_row = jnp.where(btr == b, se_ref[1, b], e_row)
    nvalid = e_row - s_row

    iota0 = jax.lax.broadcasted_iota(jnp.int32, (_CT, _RT), 0)

    for k in range(_K):
        cm = cmin_ref[...]
        m = jnp.min(cm, axis=0, keepdims=True)
        tc = jnp.full((1, _RT), big, jnp.int32)
        for c in range(nt):
            tc = jnp.minimum(
                tc, jnp.where(cm[c:c + 1, :] == m, c, big))

        def pickpass(c, idx):
            rowsel = tc == c
            tile = strip_ref[pl.ds(c * _CT, _CT), :]
            eq = (tile == m) & rowsel
            jstar = jnp.min(jnp.where(eq, iota0, big), axis=0, keepdims=True)
            newtile = jnp.where(iota0 == jstar, inf, tile)
            strip_ref[pl.ds(c * _CT, _CT), :] = newtile
            cmin_ref[pl.ds(c, 1), :] = jnp.min(newtile, axis=0, keepdims=True)
            return jnp.where(rowsel, c * _CT + jstar, idx)

        idx = jax.lax.fori_loop(c0, c1, pickpass, jnp.full((1, _RT), big))

        p = k - nvalid
        idxfix = jnp.where(p < s_row, p, e_row + (p - s_row))
        idx = jnp.where(m == inf, idxfix, idx)
        idx_ref[k:k + 1, :] = idx


def _knn(x, btc2, btr2, se, cb, d):
    n = x.shape[0]
    grid_spec = pltpu.PrefetchScalarGridSpec(
        num_scalar_prefetch=2,
        grid=(n // _RT,),
        in_specs=[
            pl.BlockSpec((_RT, d), lambda t, se, cb: (t, 0)),
            pl.BlockSpec((1, _RT), lambda t, se, cb: (0, t)),
            pl.BlockSpec((n, d), lambda t, se, cb: (0, 0)),
            pl.BlockSpec((n, 1), lambda t, se, cb: (0, 0)),
        ],
        out_specs=pl.BlockSpec((_K, _RT), lambda t, se, cb: (0, t)),
        scratch_shapes=[pltpu.VMEM((n, _RT), jnp.float32),
                        pltpu.VMEM((n // _CT, _RT), jnp.float32)],
    )
    return pl.pallas_call(
        functools.partial(_knn_body, d=d),
        grid_spec=grid_spec,
        out_shape=jax.ShapeDtypeStruct((_K, n), jnp.int32),
    )(se, cb, x, btr2, x, btc2)


def _aggr_pool_body(x_ref, batch_ref, w_ref, b_ref, out_ref):
    t = pl.program_id(0)
    h = x_ref[...] @ w_ref[...] + b_ref[...][None, :]
    bt = batch_ref[0]
    rows = []
    for b in range(_B):
        m = bt == b
        rows.append(jnp.max(jnp.where(m, h, -jnp.inf), axis=0, keepdims=True))
    acc = jnp.concatenate(rows, axis=0)

    @pl.when(t == 0)
    def _():
        out_ref[...] = acc

    @pl.when(t != 0)
    def _():
        out_ref[...] = jnp.maximum(out_ref[...], acc)


def kernel(pos, batch, b0l0_W, b0l0_b, b0l1_W, b0l1_b, b1l0_W, b1l0_b,
           aggr_W, aggr_b, h0_W, h0_b, h1_W, h1_b, h2_W, h2_b):
    batch = batch.astype(jnp.int32)

    arangeb = jnp.arange(_B, dtype=jnp.int32)
    starts = jnp.searchsorted(batch, arangeb, side="left").astype(jnp.int32)
    ends = jnp.searchsorted(batch, arangeb, side="right").astype(jnp.int32)
    se = jnp.stack([starts, ends])
    btile = batch.reshape(_N // _RT, _RT)
    c0 = starts[btile[:, 0]] // _CT
    c1 = (ends[btile[:, -1]] + _CT - 1) // _CT
    cb = jnp.stack([c0, c1], axis=1).astype(jnp.int32)
    btr2 = batch.reshape(1, _N)
    btc2 = batch.reshape(_N, 1)

    pos8 = jnp.pad(pos, ((0, 0), (0, 5)))
    idx1 = _knn(pos8, btc2, btr2, se, cb, 8)
    a1 = pos @ b0l0_W[:3] + b0l0_b
    bm1 = pos @ b0l0_W[3:]
    c1_ = a1 - bm1
    e = c1_[None, :, :] + bm1[idx1]
    h = jax.nn.relu(e) @ b0l1_W + b0l1_b
    x1 = jnp.max(h, axis=0)

    idx2 = _knn(x1, btc2, btr2, se, cb, 64)
    c2 = x1 @ (b1l0_W[:64] - b1l0_W[64:]) + b1l0_b
    d2v = x1 @ b1l0_W[64:]
    x2 = c2 + jnp.max(d2v[idx2], axis=0)

    xcat = jnp.concatenate([x1, x2], axis=1)
    bt3 = batch.reshape(_N // _RT, _RT, 1)
    pooled = pl.pallas_call(
        _aggr_pool_body,
        grid=(_N // _RT,),
        in_specs=[
            pl.BlockSpec((_RT, 192), lambda t: (t, 0)),
            pl.BlockSpec((1, _RT, 1), lambda t: (t, 0, 0)),
            pl.BlockSpec((192, 1024), lambda t: (0, 0)),
            pl.BlockSpec((1024,), lambda t: (0,)),
        ],
        out_specs=pl.BlockSpec((_B, 1024), lambda t: (0, 0)),
        out_shape=jax.ShapeDtypeStruct((_B, 1024), jnp.float32),
    )(xcat, bt3, aggr_W, aggr_b)

    h = jax.nn.relu(pooled @ h0_W + h0_b)
    h = jax.nn.relu(h @ h1_W + h1_b)
    return h @ h2_W + h2_b


# speedup vs baseline: 17.3502x; 1.7591x over previous
import functools

import jax
import jax.numpy as jnp
from jax import lax
from jax.experimental import pallas as pl
from jax.experimental.pallas import tpu as pltpu
from jax.experimental.pallas import tpu_sc as plsc

_N = 8192
_B = 8
_K = 20
_RT = 256
_CT = 256


def _knn_body(se_ref, cb_ref, xr_ref, btr_ref, xc_ref, btc_ref, idx_ref,
              strip_ref, cmin_ref, *, d):
    t = pl.program_id(0)
    c0 = cb_ref[t, 0]
    c1 = cb_ref[t, 1]
    nt = strip_ref.shape[0] // _CT

    rr = xr_ref[...]
    sq_r = jnp.sum(rr * rr, axis=1, keepdims=True)
    brmat = jnp.concatenate(
        [rr, jnp.ones((_RT, 1), jnp.float32), sq_r], 1)
    btr = btr_ref[...]

    inf = jnp.float32(jnp.inf)
    big = jnp.int32(2 * _N)
    cmin_ref[...] = jnp.full(cmin_ref.shape, inf)

    def dist_tile(c, _):
        cc = xc_ref[pl.ds(c * _CT, _CT), :]
        sqc = jnp.sum(cc * cc, axis=1, keepdims=True)
        acmat = jnp.concatenate(
            [-2.0 * cc, sqc, jnp.ones((_CT, 1), jnp.float32)], 1)
        d2 = jax.lax.dot_general(acmat, brmat, (((1,), (1,)), ((), ())),
                                 preferred_element_type=jnp.float32)
        btc = btc_ref[pl.ds(c * _CT, _CT), :]
        d2 = jnp.where(btc != btr, inf, d2)
        strip_ref[pl.ds(c * _CT, _CT), :] = d2
        cmin_ref[pl.ds(c, 1), :] = jnp.min(d2, axis=0, keepdims=True)
        return 0

    jax.lax.fori_loop(c0, c1, dist_tile, 0)

    s_row = jnp.zeros((1, _RT), jnp.int32)
    e_row = jnp.zeros((1, _RT), jnp.int32)
    for b in range(_B):
        s_row = jnp.where(btr == b, se_ref[0, b], s_row)
        e_row = jnp.where(btr == b, se_ref[1, b], e_row)
    nvalid = e_row - s_row

    iota0 = jax.lax.broadcasted_iota(jnp.int32, (_CT, _RT), 0)

    for k in range(_K):
        cm = cmin_ref[...]
        m = jnp.min(cm, axis=0, keepdims=True)
        tc = jnp.full((1, _RT), big, jnp.int32)
        for c in range(nt):
            tc = jnp.minimum(
                tc, jnp.where(cm[c:c + 1, :] == m, c, big))

        def pickpass(c, idx):
            rowsel = tc == c
            tile = strip_ref[pl.ds(c * _CT, _CT), :]
            eq = (tile == m) & rowsel
            jstar = jnp.min(jnp.where(eq, iota0, big), axis=0, keepdims=True)
            newtile = jnp.where(iota0 == jstar, inf, tile)
            strip_ref[pl.ds(c * _CT, _CT), :] = newtile
            cmin_ref[pl.ds(c, 1), :] = jnp.min(newtile, axis=0, keepdims=True)
            return jnp.where(rowsel, c * _CT + jstar, idx)

        idx = jax.lax.fori_loop(c0, c1, pickpass, jnp.full((1, _RT), big))

        p = k - nvalid
        idxfix = jnp.where(p < s_row, p, e_row + (p - s_row))
        idx = jnp.where(m == inf, idxfix, idx)
        idx_ref[k:k + 1, :] = idx


def _knn(x, btc2, btr2, se, cb, d):
    n = x.shape[0]
    grid_spec = pltpu.PrefetchScalarGridSpec(
        num_scalar_prefetch=2,
        grid=(n // _RT,),
        in_specs=[
            pl.BlockSpec((_RT, d), lambda t, se, cb: (t, 0)),
            pl.BlockSpec((1, _RT), lambda t, se, cb: (0, t)),
            pl.BlockSpec((n, d), lambda t, se, cb: (0, 0)),
            pl.BlockSpec((n, 1), lambda t, se, cb: (0, 0)),
        ],
        out_specs=pl.BlockSpec((_K, _RT), lambda t, se, cb: (0, t)),
        scratch_shapes=[pltpu.VMEM((n, _RT), jnp.float32),
                        pltpu.VMEM((n // _CT, _RT), jnp.float32)],
    )
    return pl.pallas_call(
        functools.partial(_knn_body, d=d),
        grid_spec=grid_spec,
        out_shape=jax.ShapeDtypeStruct((_K, n), jnp.int32),
    )(se, cb, x, btr2, x, btc2)


def _sc_gather(table, idxflat, d):
    m = idxflat.shape[0]
    info = plsc.get_sparse_core_info()
    nw = info.num_cores * info.num_subcores
    per_w = m // nw
    ch = 128
    nch = per_w // ch
    mesh = plsc.VectorSubcoreMesh(core_axis_name="c", subcore_axis_name="s")

    @functools.partial(
        pl.kernel, mesh=mesh,
        out_type=jax.ShapeDtypeStruct((m, d), jnp.float32),
        scratch_types=[
            pltpu.VMEM((ch,), jnp.int32),
            pltpu.VMEM((ch, d), jnp.float32),
            pltpu.SemaphoreType.DMA,
        ],
    )
    def k(table_hbm, idx_hbm, out_hbm, idx_v, rows_v, sem):
        wid = lax.axis_index("s") * info.num_cores + lax.axis_index("c")
        base = wid * per_w

        def body(q, _):
            off = base + q * ch
            pltpu.sync_copy(idx_hbm.at[pl.ds(off, ch)], idx_v)
            pltpu.async_copy(table_hbm.at[idx_v], rows_v, sem).wait()
            pltpu.sync_copy(rows_v, out_hbm.at[pl.ds(off, ch)])
            return 0

        lax.fori_loop(0, nch, body, 0)

    return k(table, idxflat)


def _conv1_body(g1_ref, cadd_ref, w1_ref, b1_ref, wc2_ref, bc2_ref, wd_ref,
                x1_ref, c2_ref, d2v_ref):
    e = jax.nn.relu(g1_ref[..., :64] + cadd_ref[...][None])
    h = lax.dot_general(e, w1_ref[...], (((2,), (0,)), ((), ())),
                        preferred_element_type=jnp.float32)
    x1 = jnp.max(h, axis=0) + b1_ref[...][None, :]
    x1_ref[...] = x1
    c2_ref[...] = x1 @ wc2_ref[...] + bc2_ref[...][None, :]
    d2v_ref[...] = x1 @ wd_ref[...]


def _aggr_body(x1_ref, c2_ref, g2_ref, batch_ref, wa1_ref, wa2_ref, ab_ref,
               h0w_ref, h0b_ref, h1w_ref, h1b_ref, h2w_ref, h2b_ref,
               out_ref, acc_ref):
    t = pl.program_id(0)
    nsteps = pl.num_programs(0)
    m2 = jnp.max(g2_ref[...], axis=0)
    x2 = c2_ref[...] + m2
    h = (x1_ref[...] @ wa1_ref[...] + x2 @ wa2_ref[...]
         + ab_ref[...][None, :])
    bt = batch_ref[0]
    rows = []
    for b in range(_B):
        rows.append(jnp.max(jnp.where(bt == b, h, -jnp.inf), axis=0,
                            keepdims=True))
    acc = jnp.concatenate(rows, axis=0)

    @pl.when(t == 0)
    def _():
        acc_ref[...] = acc

    @pl.when(t != 0)
    def _():
        acc_ref[...] = jnp.maximum(acc_ref[...], acc)

    @pl.when(t == nsteps - 1)
    def _():
        hh = jax.nn.relu(acc_ref[...] @ h0w_ref[...] + h0b_ref[...][None, :])
        hh = jax.nn.relu(hh @ h1w_ref[...] + h1b_ref[...][None, :])
        out_ref[...] = hh @ h2w_ref[...] + h2b_ref[...][None, :]


def kernel(pos, batch, b0l0_W, b0l0_b, b0l1_W, b0l1_b, b1l0_W, b1l0_b,
           aggr_W, aggr_b, h0_W, h0_b, h1_W, h1_b, h2_W, h2_b):
    batch = batch.astype(jnp.int32)

    arangeb = jnp.arange(_B, dtype=jnp.int32)
    starts = jnp.searchsorted(batch, arangeb, side="left").astype(jnp.int32)
    ends = jnp.searchsorted(batch, arangeb, side="right").astype(jnp.int32)
    se = jnp.stack([starts, ends])
    btile = batch.reshape(_N // _RT, _RT)
    c0 = starts[btile[:, 0]] // _CT
    c1 = (ends[btile[:, -1]] + _CT - 1) // _CT
    cb = jnp.stack([c0, c1], axis=1).astype(jnp.int32)
    btr2 = batch.reshape(1, _N)
    btc2 = batch.reshape(_N, 1)

    pos8 = jnp.pad(pos, ((0, 0), (0, 5)))
    idx1 = _knn(pos8, btc2, btr2, se, cb, 8)
    a1 = pos @ b0l0_W[:3] + b0l0_b
    bm1 = pos @ b0l0_W[3:]
    c1_ = a1 - bm1
    bm1p = jnp.pad(bm1, ((0, 0), (0, 64)))
    g1 = _sc_gather(bm1p, idx1.reshape(_K * _N), 128)
    x1, c2, d2v = pl.pallas_call(
        _conv1_body,
        grid=(_N // _RT,),
        in_specs=[
            pl.BlockSpec((_K, _RT, 128), lambda t: (0, t, 0)),
            pl.BlockSpec((_RT, 64), lambda t: (t, 0)),
            pl.BlockSpec((64, 64), lambda t: (0, 0)),
            pl.BlockSpec((64,), lambda t: (0,)),
            pl.BlockSpec((64, 128), lambda t: (0, 0)),
            pl.BlockSpec((128,), lambda t: (0,)),
            pl.BlockSpec((64, 128), lambda t: (0, 0)),
        ],
        out_specs=[
            pl.BlockSpec((_RT, 64), lambda t: (t, 0)),
            pl.BlockSpec((_RT, 128), lambda t: (t, 0)),
            pl.BlockSpec((_RT, 128), lambda t: (t, 0)),
        ],
        out_shape=[
            jax.ShapeDtypeStruct((_N, 64), jnp.float32),
            jax.ShapeDtypeStruct((_N, 128), jnp.float32),
            jax.ShapeDtypeStruct((_N, 128), jnp.float32),
        ],
    )(g1.reshape(_K, _N, 128), c1_, b0l1_W, b0l1_b,
      b1l0_W[:64] - b1l0_W[64:], b1l0_b, b1l0_W[64:])

    idx2 = _knn(x1, btc2, btr2, se, cb, 64)
    g2 = _sc_gather(d2v, idx2.reshape(_K * _N), 128)

    bt3 = batch.reshape(_N // _RT, _RT, 1)
    return pl.pallas_call(
        _aggr_body,
        grid=(_N // _RT,),
        in_specs=[
            pl.BlockSpec((_RT, 64), lambda t: (t, 0)),
            pl.BlockSpec((_RT, 128), lambda t: (t, 0)),
            pl.BlockSpec((_K, _RT, 128), lambda t: (0, t, 0)),
            pl.BlockSpec((1, _RT, 1), lambda t: (t, 0, 0)),
            pl.BlockSpec((64, 1024), lambda t: (0, 0)),
            pl.BlockSpec((128, 1024), lambda t: (0, 0)),
            pl.BlockSpec((1024,), lambda t: (0,)),
            pl.BlockSpec((1024, 512), lambda t: (0, 0)),
            pl.BlockSpec((512,), lambda t: (0,)),
            pl.BlockSpec((512, 256), lambda t: (0, 0)),
            pl.BlockSpec((256,), lambda t: (0,)),
            pl.BlockSpec((256, 40), lambda t: (0, 0)),
            pl.BlockSpec((40,), lambda t: (0,)),
        ],
        out_specs=pl.BlockSpec((_B, 40), lambda t: (0, 0)),
        out_shape=jax.ShapeDtypeStruct((_B, 40), jnp.float32),
        scratch_shapes=[pltpu.VMEM((_B, 1024), jnp.float32)],
    )(x1, c2, g2.reshape(_K, _N, 128), bt3, aggr_W[:64], aggr_W[64:],
      aggr_b, h0_W, h0_b, h1_W, h1_b, h2_W, h2_b)


# speedup vs baseline: 17.9582x; 1.0350x over previous
import functools

import jax
import jax.numpy as jnp
from jax import lax
from jax.experimental import pallas as pl
from jax.experimental.pallas import tpu as pltpu
from jax.experimental.pallas import tpu_sc as plsc

_N = 8192
_B = 8
_K = 20
_RT = 256
_CT = 256


def _knn_body(se_ref, cb_ref, xr_ref, btr_ref, sqr_ref, xc_ref, btc_ref,
              sqc_ref, idx_ref, strip_ref, cmin_ref, *, d):
    t = pl.program_id(0)
    c0 = cb_ref[t, 0]
    c1 = cb_ref[t, 1]
    nt = strip_ref.shape[0] // _CT

    rr = xr_ref[...]
    sqr = sqr_ref[...]
    btr = btr_ref[...]

    inf = jnp.float32(jnp.inf)
    big = jnp.int32(2 * _N)
    cmin_ref[...] = jnp.full(cmin_ref.shape, inf)

    def dist_tile(c, _):
        cc = xc_ref[pl.ds(c * _CT, _CT), :]
        g = jax.lax.dot_general(cc, rr, (((1,), (1,)), ((), ())),
                                preferred_element_type=jnp.float32)
        sqc = sqc_ref[pl.ds(c * _CT, _CT), :]
        d2 = (sqc + sqr) - 2.0 * g
        btc = btc_ref[pl.ds(c * _CT, _CT), :]
        d2 = jnp.where(btc != btr, inf, d2)
        strip_ref[pl.ds(c * _CT, _CT), :] = d2
        cmin_ref[pl.ds(c, 1), :] = jnp.min(d2, axis=0, keepdims=True)
        return 0

    jax.lax.fori_loop(c0, c1, dist_tile, 0)

    s_row = jnp.zeros((1, _RT), jnp.int32)
    e_row = jnp.zeros((1, _RT), jnp.int32)
    for b in range(_B):
        s_row = jnp.where(btr == b, se_ref[0, b], s_row)
        e_row = jnp.where(btr == b, se_ref[1, b], e_row)
    nvalid = e_row - s_row

    iota0 = jax.lax.broadcasted_iota(jnp.int32, (_CT, _RT), 0)

    for k in range(_K):
        cm = cmin_ref[...]
        m = jnp.min(cm, axis=0, keepdims=True)
        tc = jnp.full((1, _RT), big, jnp.int32)
        for c in range(nt):
            tc = jnp.minimum(
                tc, jnp.where(cm[c:c + 1, :] == m, c, big))

        def pickpass(c, idx):
            rowsel = tc == c
            tile = strip_ref[pl.ds(c * _CT, _CT), :]
            eq = (tile == m) & rowsel
            jstar = jnp.min(jnp.where(eq, iota0, big), axis=0, keepdims=True)
            newtile = jnp.where(iota0 == jstar, inf, tile)
            strip_ref[pl.ds(c * _CT, _CT), :] = newtile
            cmin_ref[pl.ds(c, 1), :] = jnp.min(newtile, axis=0, keepdims=True)
            return jnp.where(rowsel, c * _CT + jstar, idx)

        idx = jax.lax.fori_loop(c0, c1, pickpass, jnp.full((1, _RT), big))

        p = k - nvalid
        idxfix = jnp.where(p < s_row, p, e_row + (p - s_row))
        idx = jnp.where(m == inf, idxfix, idx)
        idx_ref[k:k + 1, :] = idx


def _knn(x, btc2, btr2, sq, se, cb, d):
    n = x.shape[0]
    grid_spec = pltpu.PrefetchScalarGridSpec(
        num_scalar_prefetch=2,
        grid=(n // _RT,),
        in_specs=[
            pl.BlockSpec((_RT, d), lambda t, se, cb: (t, 0)),
            pl.BlockSpec((1, _RT), lambda t, se, cb: (0, t)),
            pl.BlockSpec((1, _RT), lambda t, se, cb: (0, t)),
            pl.BlockSpec((n, d), lambda t, se, cb: (0, 0)),
            pl.BlockSpec((n, 1), lambda t, se, cb: (0, 0)),
            pl.BlockSpec((n, 1), lambda t, se, cb: (0, 0)),
        ],
        out_specs=pl.BlockSpec((_K, _RT), lambda t, se, cb: (0, t)),
        scratch_shapes=[pltpu.VMEM((n, _RT), jnp.float32),
                        pltpu.VMEM((n // _CT, _RT), jnp.float32)],
    )
    return pl.pallas_call(
        functools.partial(_knn_body, d=d),
        grid_spec=grid_spec,
        out_shape=jax.ShapeDtypeStruct((_K, n), jnp.int32),
    )(se, cb, x, btr2, sq.reshape(1, n), x, btc2, sq.reshape(n, 1))


def _sc_gather(table, idxflat, d):
    m = idxflat.shape[0]
    info = plsc.get_sparse_core_info()
    nw = info.num_cores * info.num_subcores
    per_w = m // nw
    ch = 128
    nch = per_w // ch
    mesh = plsc.VectorSubcoreMesh(core_axis_name="c", subcore_axis_name="s")

    @functools.partial(
        pl.kernel, mesh=mesh,
        out_type=jax.ShapeDtypeStruct((m, d), jnp.float32),
        scratch_types=[
            pltpu.VMEM((ch,), jnp.int32),
            pltpu.VMEM((ch, d), jnp.float32),
            pltpu.SemaphoreType.DMA,
        ],
    )
    def k(table_hbm, idx_hbm, out_hbm, idx_v, rows_v, sem):
        wid = lax.axis_index("s") * info.num_cores + lax.axis_index("c")
        base = wid * per_w

        def body(q, _):
            off = base + q * ch
            pltpu.sync_copy(idx_hbm.at[pl.ds(off, ch)], idx_v)
            pltpu.async_copy(table_hbm.at[idx_v], rows_v, sem).wait()
            pltpu.sync_copy(rows_v, out_hbm.at[pl.ds(off, ch)])
            return 0

        lax.fori_loop(0, nch, body, 0)

    return k(table, idxflat)


def _conv1_body(g1_ref, cadd_ref, w1_ref, b1_ref, wc2_ref, bc2_ref, wd_ref,
                x1_ref, c2_ref, d2v_ref):
    e = jax.nn.relu(g1_ref[..., :64] + cadd_ref[...][None])
    h = lax.dot_general(e, w1_ref[...], (((2,), (0,)), ((), ())),
                        preferred_element_type=jnp.float32)
    x1 = jnp.max(h, axis=0) + b1_ref[...][None, :]
    x1_ref[...] = x1
    c2_ref[...] = x1 @ wc2_ref[...] + bc2_ref[...][None, :]
    d2v_ref[...] = x1 @ wd_ref[...]


def _aggr_body(x1_ref, c2_ref, g2_ref, batch_ref, wa1_ref, wa2_ref, ab_ref,
               h0w_ref, h0b_ref, h1w_ref, h1b_ref, h2w_ref, h2b_ref,
               out_ref, acc_ref):
    t = pl.program_id(0)
    nsteps = pl.num_programs(0)
    m2 = jnp.max(g2_ref[...], axis=0)
    x2 = c2_ref[...] + m2
    h = (x1_ref[...] @ wa1_ref[...] + x2 @ wa2_ref[...]
         + ab_ref[...][None, :])
    bt = batch_ref[0]
    rows = []
    for b in range(_B):
        rows.append(jnp.max(jnp.where(bt == b, h, -jnp.inf), axis=0,
                            keepdims=True))
    acc = jnp.concatenate(rows, axis=0)

    @pl.when(t == 0)
    def _():
        acc_ref[...] = acc

    @pl.when(t != 0)
    def _():
        acc_ref[...] = jnp.maximum(acc_ref[...], acc)

    @pl.when(t == nsteps - 1)
    def _():
        hh = jax.nn.relu(acc_ref[...] @ h0w_ref[...] + h0b_ref[...][None, :])
        hh = jax.nn.relu(hh @ h1w_ref[...] + h1b_ref[...][None, :])
        out_ref[...] = hh @ h2w_ref[...] + h2b_ref[...][None, :]


def kernel(pos, batch, b0l0_W, b0l0_b, b0l1_W, b0l1_b, b1l0_W, b1l0_b,
           aggr_W, aggr_b, h0_W, h0_b, h1_W, h1_b, h2_W, h2_b):
    batch = batch.astype(jnp.int32)

    arangeb = jnp.arange(_B, dtype=jnp.int32)
    starts = jnp.searchsorted(batch, arangeb, side="left").astype(jnp.int32)
    ends = jnp.searchsorted(batch, arangeb, side="right").astype(jnp.int32)
    se = jnp.stack([starts, ends])
    btile = batch.reshape(_N // _RT, _RT)
    c0 = starts[btile[:, 0]] // _CT
    c1 = (ends[btile[:, -1]] + _CT - 1) // _CT
    cb = jnp.stack([c0, c1], axis=1).astype(jnp.int32)
    btr2 = batch.reshape(1, _N)
    btc2 = batch.reshape(_N, 1)

    pos8 = jnp.pad(pos, ((0, 0), (0, 5)))
    sq1 = jnp.sum(pos * pos, axis=1)
    idx1 = _knn(pos8, btc2, btr2, sq1, se, cb, 8)
    a1 = pos @ b0l0_W[:3] + b0l0_b
    bm1 = pos @ b0l0_W[3:]
    c1_ = a1 - bm1
    bm1p = jnp.pad(bm1, ((0, 0), (0, 64)))
    g1 = _sc_gather(bm1p, idx1.reshape(_K * _N), 128)
    x1, c2, d2v = pl.pallas_call(
        _conv1_body,
        grid=(_N // _RT,),
        in_specs=[
            pl.BlockSpec((_K, _RT, 128), lambda t: (0, t, 0)),
            pl.BlockSpec((_RT, 64), lambda t: (t, 0)),
            pl.BlockSpec((64, 64), lambda t: (0, 0)),
            pl.BlockSpec((64,), lambda t: (0,)),
            pl.BlockSpec((64, 128), lambda t: (0, 0)),
            pl.BlockSpec((128,), lambda t: (0,)),
            pl.BlockSpec((64, 128), lambda t: (0, 0)),
        ],
        out_specs=[
            pl.BlockSpec((_RT, 64), lambda t: (t, 0)),
            pl.BlockSpec((_RT, 128), lambda t: (t, 0)),
            pl.BlockSpec((_RT, 128), lambda t: (t, 0)),
        ],
        out_shape=[
            jax.ShapeDtypeStruct((_N, 64), jnp.float32),
            jax.ShapeDtypeStruct((_N, 128), jnp.float32),
            jax.ShapeDtypeStruct((_N, 128), jnp.float32),
        ],
    )(g1.reshape(_K, _N, 128), c1_, b0l1_W, b0l1_b,
      b1l0_W[:64] - b1l0_W[64:], b1l0_b, b1l0_W[64:])

    sq2 = jnp.sum(x1 * x1, axis=1)
    idx2 = _knn(x1, btc2, btr2, sq2, se, cb, 64)
    g2 = _sc_gather(d2v, idx2.reshape(_K * _N), 128)

    bt3 = batch.reshape(_N // _RT, _RT, 1)
    return pl.pallas_call(
        _aggr_body,
        grid=(_N // _RT,),
        in_specs=[
            pl.BlockSpec((_RT, 64), lambda t: (t, 0)),
            pl.BlockSpec((_RT, 128), lambda t: (t, 0)),
            pl.BlockSpec((_K, _RT, 128), lambda t: (0, t, 0)),
            pl.BlockSpec((1, _RT, 1), lambda t: (t, 0, 0)),
            pl.BlockSpec((64, 1024), lambda t: (0, 0)),
            pl.BlockSpec((128, 1024), lambda t: (0, 0)),
            pl.BlockSpec((1024,), lambda t: (0,)),
            pl.BlockSpec((1024, 512), lambda t: (0, 0)),
            pl.BlockSpec((512,), lambda t: (0,)),
            pl.BlockSpec((512, 256), lambda t: (0, 0)),
            pl.BlockSpec((256,), lambda t: (0,)),
            pl.BlockSpec((256, 40), lambda t: (0, 0)),
            pl.BlockSpec((40,), lambda t: (0,)),
        ],
        out_specs=pl.BlockSpec((_B, 40), lambda t: (0, 0)),
        out_shape=jax.ShapeDtypeStruct((_B, 40), jnp.float32),
        scratch_shapes=[pltpu.VMEM((_B, 1024), jnp.float32)],
    )(x1, c2, g2.reshape(_K, _N, 128), bt3, aggr_W[:64], aggr_W[64:],
      aggr_b, h0_W, h0_b, h1_W, h1_b, h2_W, h2_b)
